# Initial kernel scaffold; baseline (speedup 1.0000x reference)
import jax
import jax.numpy as jnp
from jax.experimental import pallas as pl


def kernel(x, adj_indices, adj_values, W, b):
    raise NotImplementedError("write your pallas kernel here")


# speedup vs baseline: 2.0294x; 2.0294x over previous
import functools

import jax
import jax.numpy as jnp
from jax import lax
from jax.experimental import pallas as pl
from jax.experimental.pallas import tpu as pltpu
from jax.experimental.pallas import tpu_sc as plsc

N = 10000
D = 128
C = 4
NW = 32
CHUNK = 1600
LANES = 16


def _matmul_t_body(x_ref, w_ref, b_ref, o_ref):
    o_ref[...] = lax.dot_general(
        w_ref[...], x_ref[...],
        (((1,), (1,)), ((), ())),
        preferred_element_type=jnp.float32,
    ) + b_ref[...]


def _transpose_body(i_ref, e_ref, o_ref):
    o_ref[...] = lax.dot_general(
        i_ref[...], e_ref[...],
        (((0,), (0,)), ((), ())),
        preferred_element_type=jnp.float32,
    )


def _sc_body(hT, rows, cols, vals, outT,
             h0, h1, h2, h3, a0, a1, a2, a3, rbuf, cbuf, vbuf):
    hs = (h0, h1, h2, h3)
    accs = (a0, a1, a2, a3)
    wid = lax.axis_index("s") * 2 + lax.axis_index("c")
    col0 = wid * C

    for d in range(C):
        pltpu.sync_copy(hT.at[col0 + d], hs[d])

    zero = jnp.zeros((LANES,), jnp.float32)

    def _zero_body(j, _):
        base = j * LANES
        for d in range(C):
            accs[d][pl.ds(base, LANES)] = zero
        return _

    lax.fori_loop(0, N // LANES, _zero_body, None)

    E = rows.shape[0]
    n_chunks = E // CHUNK

    def _chunk_body(k, _):
        e0 = k * CHUNK
        pltpu.sync_copy(rows.at[pl.ds(e0, CHUNK)], rbuf)
        pltpu.sync_copy(cols.at[pl.ds(e0, CHUNK)], cbuf)
        pltpu.sync_copy(vals.at[pl.ds(e0, CHUNK)], vbuf)

        def _group_body(i, _):
            base = i * LANES
            r16 = rbuf[pl.ds(base, LANES)]
            c16 = cbuf[pl.ds(base, LANES)]
            v16 = vbuf[pl.ds(base, LANES)]
            for d in range(C):
                g = plsc.load_gather(hs[d], [c16])
                plsc.addupdate_scatter(accs[d], [r16], g * v16)
            return _

        lax.fori_loop(0, CHUNK // LANES, _group_body, None)
        return _

    lax.fori_loop(0, n_chunks, _chunk_body, None)

    for d in range(C):
        pltpu.sync_copy(accs[d], outT.at[col0 + d])


def _make_sc_call():
    mesh = plsc.VectorSubcoreMesh(core_axis_name="c", subcore_axis_name="s")
    return functools.partial(
        pl.kernel,
        mesh=mesh,
        out_type=jax.ShapeDtypeStruct((D, N), jnp.float32),
        compiler_params=pltpu.CompilerParams(needs_layout_passes=False),
        scratch_types=(
            [pltpu.VMEM((N,), jnp.float32) for _ in range(2 * C)]
            + [pltpu.VMEM((CHUNK,), jnp.int32),
               pltpu.VMEM((CHUNK,), jnp.int32),
               pltpu.VMEM((CHUNK,), jnp.float32)]
        ),
    )(_sc_body)


def kernel(x, adj_indices, adj_values, W, b):
    n, d_in = x.shape
    d_out = W.shape[0]

    hiddenT = pl.pallas_call(
        _matmul_t_body,
        out_shape=jax.ShapeDtypeStruct((d_out, n), jnp.float32),
    )(x, W, b[:, None])

    sc_call = _make_sc_call()
    outT = sc_call(hiddenT, adj_indices[0], adj_indices[1], adj_values)

    eye = jnp.eye(d_out, dtype=jnp.float32)
    out = pl.pallas_call(
        _transpose_body,
        out_shape=jax.ShapeDtypeStruct((n, d_out), jnp.float32),
    )(outT, eye)
    return out


# speedup vs baseline: 3.1958x; 1.5748x over previous
import functools

import jax
import jax.numpy as jnp
from jax import lax
from jax.experimental import pallas as pl
from jax.experimental.pallas import tpu as pltpu
from jax.experimental.pallas import tpu_sc as plsc

N = 10000
D = 128
C = 4
NW = 32
CHUNK = 3200
LANES = 16
UNROLL = 4


def _matmul_t_body(x_ref, w_ref, b_ref, o_ref):
    o_ref[...] = lax.dot_general(
        w_ref[...], x_ref[...],
        (((1,), (1,)), ((), ())),
        preferred_element_type=jnp.float32,
    ) + b_ref[...]


def _transpose_body(i_ref, e_ref, o_ref):
    o_ref[...] = lax.dot_general(
        i_ref[...], e_ref[...],
        (((0,), (0,)), ((), ())),
        preferred_element_type=jnp.float32,
    )


def _sc_body(hT, rows, cols, vals, outT,
             h0, h1, h2, h3, a0, a1, a2, a3,
             rb0, cb0, vb0, rb1, cb1, vb1, sem0, sem1):
    hs = (h0, h1, h2, h3)
    accs = (a0, a1, a2, a3)
    bufs = ((rb0, cb0, vb0, sem0), (rb1, cb1, vb1, sem1))
    wid = lax.axis_index("s") * 2 + lax.axis_index("c")
    col0 = wid * C

    E = rows.shape[0]
    n_chunks = E // CHUNK

    def _issue(k, which):
        rb, cb, vb, sem = bufs[which]
        e0 = k * CHUNK
        pltpu.async_copy(rows.at[pl.ds(e0, CHUNK)], rb, sem)
        pltpu.async_copy(cols.at[pl.ds(e0, CHUNK)], cb, sem)
        pltpu.async_copy(vals.at[pl.ds(e0, CHUNK)], vb, sem)

    def _drain(which):
        rb, cb, vb, sem = bufs[which]
        pltpu.make_async_copy(rows.at[pl.ds(0, CHUNK)], rb, sem).wait()
        pltpu.make_async_copy(cols.at[pl.ds(0, CHUNK)], cb, sem).wait()
        pltpu.make_async_copy(vals.at[pl.ds(0, CHUNK)], vb, sem).wait()

    def _process(which):
        rb, cb, vb, _ = bufs[which]

        def _group_body(i, _):
            for u in range(UNROLL):
                base = (i * UNROLL + u) * LANES
                r16 = rb[pl.ds(base, LANES)]
                c16 = cb[pl.ds(base, LANES)]
                v16 = vb[pl.ds(base, LANES)]
                for d in range(C):
                    g = plsc.load_gather(hs[d], [c16])
                    plsc.addupdate_scatter(accs[d], [r16], g * v16)
            return _

        lax.fori_loop(0, CHUNK // (LANES * UNROLL), _group_body, None)

    _issue(0, 0)
    _issue(1, 1)

    for d in range(C):
        pltpu.sync_copy(hT.at[col0 + d], hs[d])

    zero = jnp.zeros((LANES,), jnp.float32)

    def _zero_body(j, _):
        base = j * LANES
        for d in range(C):
            accs[d][pl.ds(base, LANES)] = zero
        return _

    lax.fori_loop(0, N // LANES, _zero_body, None)

    def _outer(k, _):
        ca = 2 * k
        for which in range(2):
            c_cur = ca + which
            _drain(which)

            @pl.when(c_cur + 2 < n_chunks)
            def _():
                _issue(c_cur + 2, which)

            _process(which)
        return _

    lax.fori_loop(0, n_chunks // 2, _outer, None)

    for d in range(C):
        pltpu.sync_copy(accs[d], outT.at[col0 + d])


def _make_sc_call():
    mesh = plsc.VectorSubcoreMesh(core_axis_name="c", subcore_axis_name="s")
    return functools.partial(
        pl.kernel,
        mesh=mesh,
        out_type=jax.ShapeDtypeStruct((D, N), jnp.float32),
        compiler_params=pltpu.CompilerParams(needs_layout_passes=False),
        scratch_types=(
            [pltpu.VMEM((N,), jnp.float32) for _ in range(2 * C)]
            + [pltpu.VMEM((CHUNK,), jnp.int32),
               pltpu.VMEM((CHUNK,), jnp.int32),
               pltpu.VMEM((CHUNK,), jnp.float32)] * 2
            + [pltpu.SemaphoreType.DMA, pltpu.SemaphoreType.DMA]
        ),
    )(_sc_body)


def kernel(x, adj_indices, adj_values, W, b):
    n, d_in = x.shape
    d_out = W.shape[0]

    hiddenT = pl.pallas_call(
        _matmul_t_body,
        out_shape=jax.ShapeDtypeStruct((d_out, n), jnp.float32),
    )(x, W, b[:, None])

    sc_call = _make_sc_call()
    outT = sc_call(hiddenT, adj_indices[0], adj_indices[1], adj_values)

    eye = jnp.eye(d_out, dtype=jnp.float32)
    out = pl.pallas_call(
        _transpose_body,
        out_shape=jax.ShapeDtypeStruct((n, d_out), jnp.float32),
    )(outT, eye)
    return out


# speedup vs baseline: 3.1959x; 1.0000x over previous
import functools

import jax
import jax.numpy as jnp
from jax import lax
from jax.experimental import pallas as pl
from jax.experimental.pallas import tpu as pltpu
from jax.experimental.pallas import tpu_sc as plsc

N = 10000
D = 128
C = 4
NW = 32
CHUNK = 3200
LANES = 16
UNROLL = 4


def _matmul_t_body(x_ref, w_ref, b_ref, o_ref):
    o_ref[...] = lax.dot_general(
        w_ref[...], x_ref[...],
        (((1,), (1,)), ((), ())),
        preferred_element_type=jnp.float32,
    ) + b_ref[...]


def _transpose_body(i_ref, e_ref, o_ref):
    o_ref[...] = lax.dot_general(
        i_ref[...], e_ref[...],
        (((0,), (0,)), ((), ())),
        preferred_element_type=jnp.float32,
    )


def _sc_body(hT, rows, cols, vals, outT,
             h0, h1, h2, h3, a0, a1, a2, a3,
             rb0, cb0, vb0, rb1, cb1, vb1, sem0, sem1):
    hs = (h0, h1, h2, h3)
    accs = (a0, a1, a2, a3)
    bufs = ((rb0, cb0, vb0, sem0), (rb1, cb1, vb1, sem1))
    wid = lax.axis_index("s") * 2 + lax.axis_index("c")
    col0 = wid * C

    E = rows.shape[0]
    n_chunks = E // CHUNK

    def _issue(k, which):
        rb, cb, vb, sem = bufs[which]
        e0 = k * CHUNK
        pltpu.async_copy(rows.at[pl.ds(e0, CHUNK)], rb, sem)
        pltpu.async_copy(cols.at[pl.ds(e0, CHUNK)], cb, sem)
        pltpu.async_copy(vals.at[pl.ds(e0, CHUNK)], vb, sem)

    def _drain(which):
        rb, cb, vb, sem = bufs[which]
        pltpu.make_async_copy(rows.at[pl.ds(0, CHUNK)], rb, sem).wait()
        pltpu.make_async_copy(cols.at[pl.ds(0, CHUNK)], cb, sem).wait()
        pltpu.make_async_copy(vals.at[pl.ds(0, CHUNK)], vb, sem).wait()

    def _process(which):
        rb, cb, vb, _ = bufs[which]

        def _group_body(i, _):
            for u in range(UNROLL):
                base = (i * UNROLL + u) * LANES
                r16 = rb[pl.ds(base, LANES)]
                c16 = cb[pl.ds(base, LANES)]
                v16 = vb[pl.ds(base, LANES)]
                for d in range(C):
                    g = plsc.load_gather(hs[d], [c16])
                    plsc.addupdate_scatter(accs[d], [r16], g * v16)
            return _

        lax.fori_loop(0, CHUNK // (LANES * UNROLL), _group_body, None)

    _issue(0, 0)
    _issue(1, 1)

    for d in range(C):
        pltpu.sync_copy(hT.at[col0 + d], hs[d])

    zero = jnp.zeros((LANES,), jnp.float32)

    def _zero_body(j, _):
        base = j * LANES
        for d in range(C):
            accs[d][pl.ds(base, LANES)] = zero
        return _

    lax.fori_loop(0, N // LANES, _zero_body, None)

    def _outer(k, _):
        ca = 2 * k
        for which in range(2):
            c_cur = ca + which
            _drain(which)
            _process(which)

            @pl.when(c_cur + 2 < n_chunks)
            def _():
                _issue(c_cur + 2, which)
        return _

    lax.fori_loop(0, n_chunks // 2, _outer, None)

    for d in range(C):
        pltpu.sync_copy(accs[d], outT.at[col0 + d])


def _make_sc_call():
    mesh = plsc.VectorSubcoreMesh(core_axis_name="c", subcore_axis_name="s")
    return functools.partial(
        pl.kernel,
        mesh=mesh,
        out_type=jax.ShapeDtypeStruct((D, N), jnp.float32),
        compiler_params=pltpu.CompilerParams(needs_layout_passes=False),
        scratch_types=(
            [pltpu.VMEM((N,), jnp.float32) for _ in range(2 * C)]
            + [pltpu.VMEM((CHUNK,), jnp.int32),
               pltpu.VMEM((CHUNK,), jnp.int32),
               pltpu.VMEM((CHUNK,), jnp.float32)] * 2
            + [pltpu.SemaphoreType.DMA, pltpu.SemaphoreType.DMA]
        ),
    )(_sc_body)


def kernel(x, adj_indices, adj_values, W, b):
    n, d_in = x.shape
    d_out = W.shape[0]

    hiddenT = pl.pallas_call(
        _matmul_t_body,
        out_shape=jax.ShapeDtypeStruct((d_out, n), jnp.float32),
    )(x, W, b[:, None])

    sc_call = _make_sc_call()
    outT = sc_call(hiddenT, adj_indices[0], adj_indices[1], adj_values)

    eye = jnp.eye(d_out, dtype=jnp.float32)
    out = pl.pallas_call(
        _transpose_body,
        out_shape=jax.ShapeDtypeStruct((n, d_out), jnp.float32),
    )(outT, eye)
    return out


# speedup vs baseline: 8.0482x; 2.5183x over previous
import functools

import jax
import jax.numpy as jnp
from jax import lax
from jax.experimental import pallas as pl
from jax.experimental.pallas import tpu as pltpu
from jax.experimental.pallas import tpu_sc as plsc

N = 10000
D = 128
C = 4
NW = 32
CHUNK = 3200
LANES = 16
UNROLL = 4


def _matmul_t_body(x_ref, w_ref, b_ref, o_ref):
    o_ref[...] = lax.dot_general(
        w_ref[...], x_ref[...],
        (((1,), (1,)), ((), ())),
        preferred_element_type=jnp.float32,
    ) + b_ref[...]


def _transpose_body(i_ref, e_ref, o_ref):
    o_ref[...] = lax.dot_general(
        i_ref[...], e_ref[...],
        (((0,), (0,)), ((), ())),
        preferred_element_type=jnp.float32,
    )


def _sc_body(hT, rows, cols, vals, outT,
             h0, h1, h2, h3, a0, a1, a2, a3,
             rb0, cb0, vb0, rb1, cb1, vb1, sem0, sem1):
    hs = (h0, h1, h2, h3)
    accs = (a0, a1, a2, a3)
    bufs = ((rb0, cb0, vb0, sem0), (rb1, cb1, vb1, sem1))
    wid = lax.axis_index("s") * 2 + lax.axis_index("c")
    col0 = wid * C

    E = rows.shape[0]
    n_chunks = E // CHUNK

    def _issue(k, which):
        rb, cb, vb, sem = bufs[which]
        e0 = k * CHUNK
        pltpu.async_copy(rows.at[pl.ds(e0, CHUNK)], rb, sem)
        pltpu.async_copy(cols.at[pl.ds(e0, CHUNK)], cb, sem)
        pltpu.async_copy(vals.at[pl.ds(e0, CHUNK)], vb, sem)

    def _drain(which):
        rb, cb, vb, sem = bufs[which]
        pltpu.make_async_copy(rows.at[pl.ds(0, CHUNK)], rb, sem).wait()
        pltpu.make_async_copy(cols.at[pl.ds(0, CHUNK)], cb, sem).wait()
        pltpu.make_async_copy(vals.at[pl.ds(0, CHUNK)], vb, sem).wait()

    def _process(which):
        rb, cb, vb, _ = bufs[which]

        @plsc.parallel_loop(0, CHUNK // LANES, unroll=UNROLL)
        def _group_body(i):
            base = i * LANES
            r16 = rb[pl.ds(base, LANES)]
            c16 = cb[pl.ds(base, LANES)]
            v16 = vb[pl.ds(base, LANES)]
            for d in range(C):
                g = plsc.load_gather(hs[d], [c16])
                plsc.addupdate_scatter(accs[d], [r16], g * v16)

    _issue(0, 0)
    _issue(1, 1)

    for d in range(C):
        pltpu.sync_copy(hT.at[col0 + d], hs[d])

    zero = jnp.zeros((LANES,), jnp.float32)

    def _zero_body(j, _):
        base = j * LANES
        for d in range(C):
            accs[d][pl.ds(base, LANES)] = zero
        return _

    lax.fori_loop(0, N // LANES, _zero_body, None)

    def _outer(k, _):
        ca = 2 * k
        for which in range(2):
            c_cur = ca + which
            _drain(which)
            _process(which)

            @pl.when(c_cur + 2 < n_chunks)
            def _():
                _issue(c_cur + 2, which)
        return _

    lax.fori_loop(0, n_chunks // 2, _outer, None)

    for d in range(C):
        pltpu.sync_copy(accs[d], outT.at[col0 + d])


def _make_sc_call():
    mesh = plsc.VectorSubcoreMesh(core_axis_name="c", subcore_axis_name="s")
    return functools.partial(
        pl.kernel,
        mesh=mesh,
        out_type=jax.ShapeDtypeStruct((D, N), jnp.float32),
        compiler_params=pltpu.CompilerParams(needs_layout_passes=False),
        scratch_types=(
            [pltpu.VMEM((N,), jnp.float32) for _ in range(2 * C)]
            + [pltpu.VMEM((CHUNK,), jnp.int32),
               pltpu.VMEM((CHUNK,), jnp.int32),
               pltpu.VMEM((CHUNK,), jnp.float32)] * 2
            + [pltpu.SemaphoreType.DMA, pltpu.SemaphoreType.DMA]
        ),
    )(_sc_body)


def kernel(x, adj_indices, adj_values, W, b):
    n, d_in = x.shape
    d_out = W.shape[0]

    hiddenT = pl.pallas_call(
        _matmul_t_body,
        out_shape=jax.ShapeDtypeStruct((d_out, n), jnp.float32),
    )(x, W, b[:, None])

    sc_call = _make_sc_call()
    outT = sc_call(hiddenT, adj_indices[0], adj_indices[1], adj_values)

    eye = jnp.eye(d_out, dtype=jnp.float32)
    out = pl.pallas_call(
        _transpose_body,
        out_shape=jax.ShapeDtypeStruct((n, d_out), jnp.float32),
    )(outT, eye)
    return out


# speedup vs baseline: 9.5302x; 1.1841x over previous
import functools

import jax
import jax.numpy as jnp
from jax import lax
from jax.experimental import pallas as pl
from jax.experimental.pallas import tpu as pltpu
from jax.experimental.pallas import tpu_sc as plsc

N = 10000
D = 128
NW = 32
CHUNK = 3200
LANES = 16
UNROLL = 4
RC_SHIFT = 14


def _matmul_pack_body(x_ref, w_ref, b_ref, adj_ref, hp_ref, rc_ref):
    hid = lax.dot_general(
        w_ref[...], x_ref[...],
        (((1,), (1,)), ((), ())),
        preferred_element_type=jnp.float32,
    ) + b_ref[...]
    top = hid[:D // 2]
    bot = hid[D // 2:]
    tb = lax.bitcast_convert_type(top.astype(jnp.bfloat16), jnp.uint16)
    bb = lax.bitcast_convert_type(bot.astype(jnp.bfloat16), jnp.uint16)
    hp_ref[...] = (bb.astype(jnp.int32) << 16) | tb.astype(jnp.int32)
    rc_ref[...] = (adj_ref[0:1, :] << RC_SHIFT) + adj_ref[1:2, :]


def _transpose_body(i_ref, e_ref, o_ref):
    o_ref[...] = lax.dot_general(
        i_ref[...], e_ref[...],
        (((0,), (0,)), ((), ())),
        preferred_element_type=jnp.float32,
    )


def _sc_body(hp, rc, vals, outT,
             hp0, hp1, a0, a1, a2, a3,
             kb0, vb0, kb1, vb1, sem0, sem1):
    hps = (hp0, hp1)
    accs = (a0, a1, a2, a3)
    bufs = ((kb0, vb0, sem0), (kb1, vb1, sem1))
    wid = lax.axis_index("s") * 2 + lax.axis_index("c")

    E = rc.shape[0]
    n_chunks = E // CHUNK

    def _issue(k, which):
        kb, vb, sem = bufs[which]
        e0 = k * CHUNK
        pltpu.async_copy(rc.at[pl.ds(e0, CHUNK)], kb, sem)
        pltpu.async_copy(vals.at[pl.ds(e0, CHUNK)], vb, sem)

    def _drain(which):
        kb, vb, sem = bufs[which]
        pltpu.make_async_copy(rc.at[pl.ds(0, CHUNK)], kb, sem).wait()
        pltpu.make_async_copy(vals.at[pl.ds(0, CHUNK)], vb, sem).wait()

    def _process(which):
        kb, vb, _ = bufs[which]

        @plsc.parallel_loop(0, CHUNK // LANES, unroll=UNROLL)
        def _group_body(i):
            base = i * LANES
            k16 = kb[pl.ds(base, LANES)]
            v16 = vb[pl.ds(base, LANES)]
            c16 = k16 & ((1 << RC_SHIFT) - 1)
            r16 = lax.shift_right_logical(k16, RC_SHIFT)
            for j in range(2):
                g = plsc.load_gather(hps[j], [c16])
                topf = plsc.bitcast(g << 16, jnp.float32)
                botf = plsc.bitcast(g & jnp.int32(-65536), jnp.float32)
                plsc.addupdate_scatter(accs[2 * j], [r16], topf * v16)
                plsc.addupdate_scatter(accs[2 * j + 1], [r16], botf * v16)

    _issue(0, 0)
    _issue(1, 1)

    for j in range(2):
        pltpu.sync_copy(hp.at[2 * wid + j], hps[j])

    zero = jnp.zeros((LANES,), jnp.float32)

    @plsc.parallel_loop(0, N // LANES, unroll=4)
    def _zero_body(i):
        base = i * LANES
        for d in range(4):
            accs[d][pl.ds(base, LANES)] = zero

    def _outer(k, _):
        ca = 2 * k
        for which in range(2):
            c_cur = ca + which
            _drain(which)
            _process(which)

            @pl.when(c_cur + 2 < n_chunks)
            def _():
                _issue(c_cur + 2, which)
        return _

    lax.fori_loop(0, n_chunks // 2, _outer, None)

    rows_out = (2 * wid, D // 2 + 2 * wid, 2 * wid + 1, D // 2 + 2 * wid + 1)
    for a, r in zip(accs, rows_out):
        pltpu.sync_copy(a, outT.at[r])


def _make_sc_call():
    mesh = plsc.VectorSubcoreMesh(core_axis_name="c", subcore_axis_name="s")
    return functools.partial(
        pl.kernel,
        mesh=mesh,
        out_type=jax.ShapeDtypeStruct((D, N), jnp.float32),
        compiler_params=pltpu.CompilerParams(needs_layout_passes=False),
        scratch_types=(
            [pltpu.VMEM((N,), jnp.int32) for _ in range(2)]
            + [pltpu.VMEM((N,), jnp.float32) for _ in range(4)]
            + [pltpu.VMEM((CHUNK,), jnp.int32),
               pltpu.VMEM((CHUNK,), jnp.float32)] * 2
            + [pltpu.SemaphoreType.DMA, pltpu.SemaphoreType.DMA]
        ),
    )(_sc_body)


def kernel(x, adj_indices, adj_values, W, b):
    n, d_in = x.shape
    d_out = W.shape[0]
    e = adj_values.shape[0]

    hp, rc2 = pl.pallas_call(
        _matmul_pack_body,
        out_shape=(
            jax.ShapeDtypeStruct((d_out // 2, n), jnp.int32),
            jax.ShapeDtypeStruct((1, e), jnp.int32),
        ),
    )(x, W, b[:, None], adj_indices)

    sc_call = _make_sc_call()
    outT = sc_call(hp, rc2.reshape(e), adj_values)

    eye = jnp.eye(d_out, dtype=jnp.float32)
    out = pl.pallas_call(
        _transpose_body,
        out_shape=jax.ShapeDtypeStruct((n, d_out), jnp.float32),
    )(outT, eye)
    return out
